# baseline (speedup 1.0000x reference)
import functools

import jax
import jax.numpy as jnp
from jax import lax
from jax.experimental import pallas as pl
from jax.experimental.pallas import tpu as pltpu
from jax.experimental.pallas import tpu_sc as plsc

N = 10000
E = 320000
F_IN = 128
H = 64
G = 64

NC = 2
NS = 16
NW = NC * NS
EPW = E // NW
CHUNK = 125
NCHUNK = EPW // CHUNK
NA = 10240
RPS = NA // NS

KB = 8
SKEW = 4

R = 1000


def _sc_mesh():
    return plsc.VectorSubcoreMesh(core_axis_name="c", subcore_axis_name="s")


_SC_PARAMS = pltpu.CompilerParams(use_tc_tiling_on_sc=False)


def _sc_degree(dst3, zeros16):
    @functools.partial(
        pl.kernel,
        out_type=jax.ShapeDtypeStruct((NC, NA, 16), jnp.float32),
        mesh=_sc_mesh(),
        compiler_params=_SC_PARAMS,
        scratch_types=[
            pltpu.VMEM((NCHUNK, CHUNK), jnp.int32),
            pltpu.VMEM((CHUNK, 16), jnp.float32),
            pltpu.VMEM_SHARED((NA, 16), jnp.float32),
        ],
    )
    def k(dst_hbm, zeros_hbm, out_hbm, dstv, ones_v, acc):
        c = lax.axis_index("c")
        s = lax.axis_index("s")
        wid = c * NS + s
        pltpu.sync_copy(dst_hbm.at[wid], dstv)

        @pl.loop(0, CHUNK)
        def _(i):
            ones_v.at[i][...] = jnp.full((16,), 1.0, jnp.float32)

        pltpu.sync_copy(
            zeros_hbm.at[pl.ds(s * RPS, RPS)], acc.at[pl.ds(s * RPS, RPS)]
        )
        plsc.subcore_barrier()

        @pl.loop(0, NCHUNK)
        def _(j):
            pltpu.sync_copy(ones_v, acc.at[dstv.at[j]], add=True)

        plsc.subcore_barrier()
        pltpu.sync_copy(
            acc.at[pl.ds(s * RPS, RPS)], out_hbm.at[c, pl.ds(s * RPS, RPS)]
        )

    return k(dst3, zeros16)


def _sc_aggregate(hp, src3, dst3, zeros64):
    @functools.partial(
        pl.kernel,
        out_type=jax.ShapeDtypeStruct((NC, NA, H), jnp.float32),
        mesh=_sc_mesh(),
        compiler_params=_SC_PARAMS,
        scratch_types=[
            pltpu.VMEM((NCHUNK, CHUNK), jnp.int32),
            pltpu.VMEM((NCHUNK, CHUNK), jnp.int32),
            pltpu.VMEM((KB, CHUNK, H), jnp.float32),
            pltpu.VMEM_SHARED((NA, H), jnp.float32),
        ]
        + [pltpu.SemaphoreType.DMA] * (2 * KB),
    )
    def k(hp_hbm, src_hbm, dst_hbm, zeros_hbm, out_hbm, srcv, dstv, rows, acc, *sems):
        gsem = sems[:KB]
        ssem = sems[KB:]
        c = lax.axis_index("c")
        s = lax.axis_index("s")
        wid = c * NS + s
        pltpu.sync_copy(src_hbm.at[wid], srcv)
        pltpu.sync_copy(dst_hbm.at[wid], dstv)
        pltpu.sync_copy(
            zeros_hbm.at[pl.ds(s * RPS, RPS)], acc.at[pl.ds(s * RPS, RPS)]
        )
        plsc.subcore_barrier()

        @pl.loop(0, NCHUNK, step=KB)
        def _(j):
            for b in range(KB):
                @pl.when(j + b >= KB)
                def _():
                    pltpu.make_async_copy(
                        rows.at[b], acc.at[dstv.at[j + b - KB]], ssem[b]
                    ).wait()

                pltpu.async_copy(hp_hbm.at[srcv.at[j + b]], rows.at[b], gsem[b])

                bd = (b - SKEW) % KB

                @pl.when(j + b >= SKEW)
                def _():
                    pltpu.make_async_copy(
                        hp_hbm.at[srcv.at[j + b - SKEW]], rows.at[bd], gsem[bd]
                    ).wait()
                    pltpu.async_copy(
                        rows.at[bd],
                        acc.at[dstv.at[j + b - SKEW]],
                        ssem[bd],
                        add=True,
                    )

        for t in range(SKEW):
            jj = NCHUNK - SKEW + t
            b = jj % KB
            pltpu.make_async_copy(
                hp_hbm.at[srcv.at[jj]], rows.at[b], gsem[b]
            ).wait()
            pltpu.async_copy(rows.at[b], acc.at[dstv.at[jj]], ssem[b], add=True)
        for b in range(KB):
            jj = NCHUNK - KB + b
            pltpu.make_async_copy(
                rows.at[b], acc.at[dstv.at[jj]], ssem[b]
            ).wait()

        plsc.subcore_barrier()
        pltpu.sync_copy(
            acc.at[pl.ds(s * RPS, RPS)], out_hbm.at[c, pl.ds(s * RPS, RPS)]
        )

    return k(hp, src3, dst3, zeros64)


def _tc_prep(degp, x, W1):
    def body(degp_ref, x_ref, w_ref, dinv_ref, hp_ref):
        d = degp_ref[0, :, 0:1] + degp_ref[1, :, 0:1] + 1.0
        dinv = 1.0 / jnp.sqrt(d)
        dinv_ref[...] = jnp.broadcast_to(dinv, (R, H))
        h1 = jnp.dot(x_ref[...], w_ref[...], preferred_element_type=jnp.float32)
        hp_ref[...] = h1 * dinv

    return pl.pallas_call(
        body,
        grid=(N // R,),
        in_specs=[
            pl.BlockSpec((NC, R, 16), lambda i: (0, i, 0)),
            pl.BlockSpec((R, F_IN), lambda i: (i, 0)),
            pl.BlockSpec((F_IN, H), lambda i: (0, 0)),
        ],
        out_specs=[
            pl.BlockSpec((R, H), lambda i: (i, 0)),
            pl.BlockSpec((R, H), lambda i: (i, 0)),
        ],
        out_shape=[
            jax.ShapeDtypeStruct((N, H), jnp.float32),
            jax.ShapeDtypeStruct((N, H), jnp.float32),
        ],
    )(degp, x, W1)


def _tc_mid(S1, hp1, dinv, b1, W2):
    def body(S_ref, hp_ref, dinv_ref, b_ref, w_ref, o_ref):
        z = dinv_ref[...] * (S_ref[0] + S_ref[1] + hp_ref[...]) + b_ref[...]
        z = jnp.maximum(z, 0.0)
        o_ref[...] = dinv_ref[...] * jnp.dot(
            z, w_ref[...], preferred_element_type=jnp.float32
        )

    return pl.pallas_call(
        body,
        grid=(N // R,),
        in_specs=[
            pl.BlockSpec((NC, R, H), lambda i: (0, i, 0)),
            pl.BlockSpec((R, H), lambda i: (i, 0)),
            pl.BlockSpec((R, H), lambda i: (i, 0)),
            pl.BlockSpec((1, H), lambda i: (0, 0)),
            pl.BlockSpec((H, H), lambda i: (0, 0)),
        ],
        out_specs=pl.BlockSpec((R, H), lambda i: (i, 0)),
        out_shape=jax.ShapeDtypeStruct((N, H), jnp.float32),
    )(S1, hp1, dinv, b1, W2)


def _tc_final(S2, hp2, dinv, b2, batch2, W3, b3, W4, b4):
    def body(
        S_ref, hp_ref, dinv_ref, b2_ref, bat_ref, w3_ref, b3_ref, w4_ref,
        b4_ref, o_ref, psum, cnt,
    ):
        i = pl.program_id(0)

        @pl.when(i == 0)
        def _():
            psum[...] = jnp.zeros((G, H), jnp.float32)
            cnt[...] = jnp.zeros((G, 1), jnp.float32)

        z = dinv_ref[...] * (S_ref[0] + S_ref[1] + hp_ref[...]) + b2_ref[...]
        z = jnp.maximum(z, 0.0)
        gid = lax.broadcasted_iota(jnp.int32, (R, G), 1)
        onehot = (bat_ref[...] == gid).astype(jnp.float32)
        psum[...] += lax.dot_general(
            onehot, z, (((0,), (0,)), ((), ())),
            preferred_element_type=jnp.float32,
            precision=lax.Precision.HIGHEST,
        )
        cnt[...] += lax.dot_general(
            onehot, jnp.ones((R, 1), jnp.float32), (((0,), (0,)), ((), ())),
            preferred_element_type=jnp.float32,
        )

        @pl.when(i == pl.num_programs(0) - 1)
        def _():
            p = psum[...] / jnp.maximum(cnt[...], 1.0)
            r = jnp.maximum(
                jnp.dot(p, w3_ref[...], preferred_element_type=jnp.float32)
                + b3_ref[...],
                0.0,
            )
            o_ref[...] = (
                jnp.dot(r, w4_ref[...], preferred_element_type=jnp.float32)
                + b4_ref[...]
            )

    return pl.pallas_call(
        body,
        grid=(N // R,),
        in_specs=[
            pl.BlockSpec((NC, R, H), lambda i: (0, i, 0)),
            pl.BlockSpec((R, H), lambda i: (i, 0)),
            pl.BlockSpec((R, H), lambda i: (i, 0)),
            pl.BlockSpec((1, H), lambda i: (0, 0)),
            pl.BlockSpec((R, 1), lambda i: (i, 0)),
            pl.BlockSpec((H, 128), lambda i: (0, 0)),
            pl.BlockSpec((1, 128), lambda i: (0, 0)),
            pl.BlockSpec((128, 1), lambda i: (0, 0)),
            pl.BlockSpec((1, 1), lambda i: (0, 0)),
        ],
        out_specs=pl.BlockSpec((G, 1), lambda i: (0, 0)),
        out_shape=jax.ShapeDtypeStruct((G, 1), jnp.float32),
        scratch_shapes=[
            pltpu.VMEM((G, H), jnp.float32),
            pltpu.VMEM((G, 1), jnp.float32),
        ],
    )(S2, hp2, dinv, b2, batch2, W3, b3, W4, b4)


def kernel(x, edge_index, batch, W1, b1, W2, b2, W3, b3, W4, b4):
    src3 = edge_index[0].reshape(NW, NCHUNK, CHUNK)
    dst3 = edge_index[1].reshape(NW, NCHUNK, CHUNK)
    zeros16 = jnp.zeros((NA, 16), jnp.float32)
    zeros64 = jnp.zeros((NA, H), jnp.float32)
    batch2 = batch.reshape(N, 1)
    b1r = b1.reshape(1, H)
    b2r = b2.reshape(1, H)
    b3r = b3.reshape(1, 128)
    b4r = b4.reshape(1, 1)

    degp = _sc_degree(dst3, zeros16)
    dinv, hp1 = _tc_prep(degp, x, W1)
    S1 = _sc_aggregate(hp1, src3, dst3, zeros64)
    hp2 = _tc_mid(S1, hp1, dinv, b1r, W2)
    S2 = _sc_aggregate(hp2, src3, dst3, zeros64)
    out = _tc_final(S2, hp2, dinv, b2r, batch2, W3, b3r, W4, b4r)
    return jnp.squeeze(out)


# speedup vs baseline: 1.0426x; 1.0426x over previous
import functools

import jax
import jax.numpy as jnp
from jax import lax
from jax.experimental import pallas as pl
from jax.experimental.pallas import tpu as pltpu
from jax.experimental.pallas import tpu_sc as plsc

N = 10000
E = 320000
F_IN = 128
H = 64
G = 64

NC = 2
NS = 16
NW = NC * NS
EPW = E // NW
CHUNK = 125
NCHUNK = EPW // CHUNK
NA = 10240
RPS = NA // NS

KB = 8
SKEW = 4

R = 2000


def _sc_mesh():
    return plsc.VectorSubcoreMesh(core_axis_name="c", subcore_axis_name="s")


_SC_PARAMS = pltpu.CompilerParams(use_tc_tiling_on_sc=False)


def _sc_degree(dst3, zeros16):
    @functools.partial(
        pl.kernel,
        out_type=jax.ShapeDtypeStruct((NC, NA, 16), jnp.float32),
        mesh=_sc_mesh(),
        compiler_params=_SC_PARAMS,
        scratch_types=[
            pltpu.VMEM((NCHUNK, CHUNK), jnp.int32),
            pltpu.VMEM((CHUNK, 16), jnp.float32),
            pltpu.VMEM_SHARED((NA, 16), jnp.float32),
        ]
        + [pltpu.SemaphoreType.DMA] * KB,
    )
    def k(dst_hbm, zeros_hbm, out_hbm, dstv, ones_v, acc, *ssem):
        c = lax.axis_index("c")
        s = lax.axis_index("s")
        wid = c * NS + s
        pltpu.sync_copy(dst_hbm.at[wid], dstv)

        @pl.loop(0, CHUNK)
        def _(i):
            ones_v.at[i][...] = jnp.full((16,), 1.0, jnp.float32)

        pltpu.sync_copy(
            zeros_hbm.at[pl.ds(s * RPS, RPS)], acc.at[pl.ds(s * RPS, RPS)]
        )
        plsc.subcore_barrier()

        @pl.loop(0, NCHUNK, step=KB)
        def _(j):
            for b in range(KB):
                @pl.when(j + b >= KB)
                def _():
                    pltpu.make_async_copy(
                        ones_v, acc.at[dstv.at[j + b - KB]], ssem[b]
                    ).wait()

                pltpu.async_copy(ones_v, acc.at[dstv.at[j + b]], ssem[b], add=True)

        for b in range(KB):
            pltpu.make_async_copy(
                ones_v, acc.at[dstv.at[NCHUNK - KB + b]], ssem[b]
            ).wait()

        plsc.subcore_barrier()
        pltpu.sync_copy(
            acc.at[pl.ds(s * RPS, RPS)], out_hbm.at[c, pl.ds(s * RPS, RPS)]
        )

    return k(dst3, zeros16)


def _sc_aggregate(hp, src3, dst3, zeros64):
    @functools.partial(
        pl.kernel,
        out_type=jax.ShapeDtypeStruct((NC, NA, H), jnp.float32),
        mesh=_sc_mesh(),
        compiler_params=_SC_PARAMS,
        scratch_types=[
            pltpu.VMEM((NCHUNK, CHUNK), jnp.int32),
            pltpu.VMEM((NCHUNK, CHUNK), jnp.int32),
            pltpu.VMEM((KB, CHUNK, H), jnp.float32),
            pltpu.VMEM_SHARED((NA, H), jnp.float32),
        ]
        + [pltpu.SemaphoreType.DMA] * (2 * KB),
    )
    def k(hp_hbm, src_hbm, dst_hbm, zeros_hbm, out_hbm, srcv, dstv, rows, acc, *sems):
        gsem = sems[:KB]
        ssem = sems[KB:]
        c = lax.axis_index("c")
        s = lax.axis_index("s")
        wid = c * NS + s
        pltpu.sync_copy(src_hbm.at[wid], srcv)
        pltpu.sync_copy(dst_hbm.at[wid], dstv)
        pltpu.sync_copy(
            zeros_hbm.at[pl.ds(s * RPS, RPS)], acc.at[pl.ds(s * RPS, RPS)]
        )
        plsc.subcore_barrier()

        @pl.loop(0, NCHUNK, step=KB)
        def _(j):
            for b in range(KB):
                @pl.when(j + b >= KB)
                def _():
                    pltpu.make_async_copy(
                        rows.at[b], acc.at[dstv.at[j + b - KB]], ssem[b]
                    ).wait()

                pltpu.async_copy(hp_hbm.at[srcv.at[j + b]], rows.at[b], gsem[b])

                bd = (b - SKEW) % KB

                @pl.when(j + b >= SKEW)
                def _():
                    pltpu.make_async_copy(
                        hp_hbm.at[srcv.at[j + b - SKEW]], rows.at[bd], gsem[bd]
                    ).wait()
                    pltpu.async_copy(
                        rows.at[bd],
                        acc.at[dstv.at[j + b - SKEW]],
                        ssem[bd],
                        add=True,
                    )

        for t in range(SKEW):
            jj = NCHUNK - SKEW + t
            b = jj % KB
            pltpu.make_async_copy(
                hp_hbm.at[srcv.at[jj]], rows.at[b], gsem[b]
            ).wait()
            pltpu.async_copy(rows.at[b], acc.at[dstv.at[jj]], ssem[b], add=True)
        for b in range(KB):
            jj = NCHUNK - KB + b
            pltpu.make_async_copy(
                rows.at[b], acc.at[dstv.at[jj]], ssem[b]
            ).wait()

        plsc.subcore_barrier()
        pltpu.sync_copy(
            acc.at[pl.ds(s * RPS, RPS)], out_hbm.at[c, pl.ds(s * RPS, RPS)]
        )

    return k(hp, src3, dst3, zeros64)


def _tc_prep(degp, x, W1):
    def body(degp_ref, x_ref, w_ref, dinv_ref, hp_ref):
        d = degp_ref[0, :, 0:1] + degp_ref[1, :, 0:1] + 1.0
        dinv = 1.0 / jnp.sqrt(d)
        dinv_ref[...] = jnp.broadcast_to(dinv, (R, H))
        h1 = jnp.dot(x_ref[...], w_ref[...], preferred_element_type=jnp.float32)
        hp_ref[...] = h1 * dinv

    return pl.pallas_call(
        body,
        grid=(N // R,),
        in_specs=[
            pl.BlockSpec((NC, R, 16), lambda i: (0, i, 0)),
            pl.BlockSpec((R, F_IN), lambda i: (i, 0)),
            pl.BlockSpec((F_IN, H), lambda i: (0, 0)),
        ],
        out_specs=[
            pl.BlockSpec((R, H), lambda i: (i, 0)),
            pl.BlockSpec((R, H), lambda i: (i, 0)),
        ],
        out_shape=[
            jax.ShapeDtypeStruct((N, H), jnp.float32),
            jax.ShapeDtypeStruct((N, H), jnp.float32),
        ],
    )(degp, x, W1)


def _tc_mid(S1, hp1, dinv, b1, W2):
    def body(S_ref, hp_ref, dinv_ref, b_ref, w_ref, o_ref):
        z = dinv_ref[...] * (S_ref[0] + S_ref[1] + hp_ref[...]) + b_ref[...]
        z = jnp.maximum(z, 0.0)
        o_ref[...] = dinv_ref[...] * jnp.dot(
            z, w_ref[...], preferred_element_type=jnp.float32
        )

    return pl.pallas_call(
        body,
        grid=(N // R,),
        in_specs=[
            pl.BlockSpec((NC, R, H), lambda i: (0, i, 0)),
            pl.BlockSpec((R, H), lambda i: (i, 0)),
            pl.BlockSpec((R, H), lambda i: (i, 0)),
            pl.BlockSpec((1, H), lambda i: (0, 0)),
            pl.BlockSpec((H, H), lambda i: (0, 0)),
        ],
        out_specs=pl.BlockSpec((R, H), lambda i: (i, 0)),
        out_shape=jax.ShapeDtypeStruct((N, H), jnp.float32),
    )(S1, hp1, dinv, b1, W2)


def _tc_final(S2, hp2, dinv, b2, batch2, W3, b3, W4, b4):
    def body(
        S_ref, hp_ref, dinv_ref, b2_ref, bat_ref, w3_ref, b3_ref, w4_ref,
        b4_ref, o_ref, psum, cnt,
    ):
        i = pl.program_id(0)

        @pl.when(i == 0)
        def _():
            psum[...] = jnp.zeros((G, H), jnp.float32)
            cnt[...] = jnp.zeros((G, 1), jnp.float32)

        z = dinv_ref[...] * (S_ref[0] + S_ref[1] + hp_ref[...]) + b2_ref[...]
        z = jnp.maximum(z, 0.0)
        gid = lax.broadcasted_iota(jnp.int32, (R, G), 1)
        onehot = (bat_ref[...] == gid).astype(jnp.float32)
        psum[...] += lax.dot_general(
            onehot, z, (((0,), (0,)), ((), ())),
            preferred_element_type=jnp.float32,
            precision=lax.Precision.HIGHEST,
        )
        cnt[...] += lax.dot_general(
            onehot, jnp.ones((R, 1), jnp.float32), (((0,), (0,)), ((), ())),
            preferred_element_type=jnp.float32,
        )

        @pl.when(i == pl.num_programs(0) - 1)
        def _():
            p = psum[...] / jnp.maximum(cnt[...], 1.0)
            r = jnp.maximum(
                jnp.dot(p, w3_ref[...], preferred_element_type=jnp.float32)
                + b3_ref[...],
                0.0,
            )
            o_ref[...] = (
                jnp.dot(r, w4_ref[...], preferred_element_type=jnp.float32)
                + b4_ref[...]
            )

    return pl.pallas_call(
        body,
        grid=(N // R,),
        in_specs=[
            pl.BlockSpec((NC, R, H), lambda i: (0, i, 0)),
            pl.BlockSpec((R, H), lambda i: (i, 0)),
            pl.BlockSpec((R, H), lambda i: (i, 0)),
            pl.BlockSpec((1, H), lambda i: (0, 0)),
            pl.BlockSpec((R, 1), lambda i: (i, 0)),
            pl.BlockSpec((H, 128), lambda i: (0, 0)),
            pl.BlockSpec((1, 128), lambda i: (0, 0)),
            pl.BlockSpec((128, 1), lambda i: (0, 0)),
            pl.BlockSpec((1, 1), lambda i: (0, 0)),
        ],
        out_specs=pl.BlockSpec((G, 1), lambda i: (0, 0)),
        out_shape=jax.ShapeDtypeStruct((G, 1), jnp.float32),
        scratch_shapes=[
            pltpu.VMEM((G, H), jnp.float32),
            pltpu.VMEM((G, 1), jnp.float32),
        ],
    )(S2, hp2, dinv, b2, batch2, W3, b3, W4, b4)


def kernel(x, edge_index, batch, W1, b1, W2, b2, W3, b3, W4, b4):
    src3 = edge_index[0].reshape(NW, NCHUNK, CHUNK)
    dst3 = edge_index[1].reshape(NW, NCHUNK, CHUNK)
    zeros16 = jnp.zeros((NA, 16), jnp.float32)
    zeros64 = jnp.zeros((NA, H), jnp.float32)
    batch2 = batch.reshape(N, 1)
    b1r = b1.reshape(1, H)
    b2r = b2.reshape(1, H)
    b3r = b3.reshape(1, 128)
    b4r = b4.reshape(1, 1)

    degp = _sc_degree(dst3, zeros16)
    dinv, hp1 = _tc_prep(degp, x, W1)
    S1 = _sc_aggregate(hp1, src3, dst3, zeros64)
    hp2 = _tc_mid(S1, hp1, dinv, b1r, W2)
    S2 = _sc_aggregate(hp2, src3, dst3, zeros64)
    out = _tc_final(S2, hp2, dinv, b2r, batch2, W3, b3r, W4, b4r)
    return jnp.squeeze(out)


# speedup vs baseline: 1.0477x; 1.0050x over previous
import functools

import jax
import jax.numpy as jnp
from jax import lax
from jax.experimental import pallas as pl
from jax.experimental.pallas import tpu as pltpu
from jax.experimental.pallas import tpu_sc as plsc

N = 10000
E = 320000
F_IN = 128
H = 64
G = 64

NC = 2
NS = 16
NW = NC * NS
EPW = E // NW
CHUNK = 100
NCHUNK = EPW // CHUNK
NA = 10240
RPS = NA // NS

KB = 10
SKEW = 5

R = 2000


def _sc_mesh():
    return plsc.VectorSubcoreMesh(core_axis_name="c", subcore_axis_name="s")


_SC_PARAMS = pltpu.CompilerParams(use_tc_tiling_on_sc=False)


def _sc_degree(dst3, zeros16):
    @functools.partial(
        pl.kernel,
        out_type=jax.ShapeDtypeStruct((NC, NA, 16), jnp.float32),
        mesh=_sc_mesh(),
        compiler_params=_SC_PARAMS,
        scratch_types=[
            pltpu.VMEM((NCHUNK, CHUNK), jnp.int32),
            pltpu.VMEM((CHUNK, 16), jnp.float32),
            pltpu.VMEM_SHARED((NA, 16), jnp.float32),
        ]
        + [pltpu.SemaphoreType.DMA] * KB,
    )
    def k(dst_hbm, zeros_hbm, out_hbm, dstv, ones_v, acc, *ssem):
        c = lax.axis_index("c")
        s = lax.axis_index("s")
        wid = c * NS + s
        pltpu.sync_copy(dst_hbm.at[wid], dstv)

        @pl.loop(0, CHUNK)
        def _(i):
            ones_v.at[i][...] = jnp.full((16,), 1.0, jnp.float32)

        pltpu.sync_copy(
            zeros_hbm.at[pl.ds(s * RPS, RPS)], acc.at[pl.ds(s * RPS, RPS)]
        )
        plsc.subcore_barrier()

        @pl.loop(0, NCHUNK, step=KB)
        def _(j):
            for b in range(KB):
                @pl.when(j + b >= KB)
                def _():
                    pltpu.make_async_copy(
                        ones_v, acc.at[dstv.at[j + b - KB]], ssem[b]
                    ).wait()

                pltpu.async_copy(ones_v, acc.at[dstv.at[j + b]], ssem[b], add=True)

        for b in range(KB):
            pltpu.make_async_copy(
                ones_v, acc.at[dstv.at[NCHUNK - KB + b]], ssem[b]
            ).wait()

        plsc.subcore_barrier()
        pltpu.sync_copy(
            acc.at[pl.ds(s * RPS, RPS)], out_hbm.at[c, pl.ds(s * RPS, RPS)]
        )

    return k(dst3, zeros16)


def _sc_aggregate(hp, src3, dst3, zeros64):
    @functools.partial(
        pl.kernel,
        out_type=jax.ShapeDtypeStruct((NC, NA, H), jnp.float32),
        mesh=_sc_mesh(),
        compiler_params=_SC_PARAMS,
        scratch_types=[
            pltpu.VMEM((NCHUNK, CHUNK), jnp.int32),
            pltpu.VMEM((NCHUNK, CHUNK), jnp.int32),
            pltpu.VMEM((KB, CHUNK, H), jnp.float32),
            pltpu.VMEM_SHARED((NA, H), jnp.float32),
        ]
        + [pltpu.SemaphoreType.DMA] * (2 * KB),
    )
    def k(hp_hbm, src_hbm, dst_hbm, zeros_hbm, out_hbm, srcv, dstv, rows, acc, *sems):
        gsem = sems[:KB]
        ssem = sems[KB:]
        c = lax.axis_index("c")
        s = lax.axis_index("s")
        wid = c * NS + s
        pltpu.sync_copy(src_hbm.at[wid], srcv)
        pltpu.sync_copy(dst_hbm.at[wid], dstv)
        pltpu.sync_copy(
            zeros_hbm.at[pl.ds(s * RPS, RPS)], acc.at[pl.ds(s * RPS, RPS)]
        )
        plsc.subcore_barrier()

        @pl.loop(0, NCHUNK, step=KB)
        def _(j):
            for b in range(KB):
                @pl.when(j + b >= KB)
                def _():
                    pltpu.make_async_copy(
                        rows.at[b], acc.at[dstv.at[j + b - KB]], ssem[b]
                    ).wait()

                pltpu.async_copy(hp_hbm.at[srcv.at[j + b]], rows.at[b], gsem[b])

                bd = (b - SKEW) % KB

                @pl.when(j + b >= SKEW)
                def _():
                    pltpu.make_async_copy(
                        hp_hbm.at[srcv.at[j + b - SKEW]], rows.at[bd], gsem[bd]
                    ).wait()
                    pltpu.async_copy(
                        rows.at[bd],
                        acc.at[dstv.at[j + b - SKEW]],
                        ssem[bd],
                        add=True,
                    )

        for t in range(SKEW):
            jj = NCHUNK - SKEW + t
            b = jj % KB
            pltpu.make_async_copy(
                hp_hbm.at[srcv.at[jj]], rows.at[b], gsem[b]
            ).wait()
            pltpu.async_copy(rows.at[b], acc.at[dstv.at[jj]], ssem[b], add=True)
        for b in range(KB):
            jj = NCHUNK - KB + b
            pltpu.make_async_copy(
                rows.at[b], acc.at[dstv.at[jj]], ssem[b]
            ).wait()

        plsc.subcore_barrier()
        pltpu.sync_copy(
            acc.at[pl.ds(s * RPS, RPS)], out_hbm.at[c, pl.ds(s * RPS, RPS)]
        )

    return k(hp, src3, dst3, zeros64)


def _tc_prep(degp, x, W1):
    def body(degp_ref, x_ref, w_ref, dinv_ref, hp_ref):
        d = degp_ref[0, :, 0:1] + degp_ref[1, :, 0:1] + 1.0
        dinv = 1.0 / jnp.sqrt(d)
        dinv_ref[...] = jnp.broadcast_to(dinv, (R, H))
        h1 = jnp.dot(x_ref[...], w_ref[...], preferred_element_type=jnp.float32)
        hp_ref[...] = h1 * dinv

    return pl.pallas_call(
        body,
        grid=(N // R,),
        in_specs=[
            pl.BlockSpec((NC, R, 16), lambda i: (0, i, 0)),
            pl.BlockSpec((R, F_IN), lambda i: (i, 0)),
            pl.BlockSpec((F_IN, H), lambda i: (0, 0)),
        ],
        out_specs=[
            pl.BlockSpec((R, H), lambda i: (i, 0)),
            pl.BlockSpec((R, H), lambda i: (i, 0)),
        ],
        out_shape=[
            jax.ShapeDtypeStruct((N, H), jnp.float32),
            jax.ShapeDtypeStruct((N, H), jnp.float32),
        ],
    )(degp, x, W1)


def _tc_mid(S1, hp1, dinv, b1, W2):
    def body(S_ref, hp_ref, dinv_ref, b_ref, w_ref, o_ref):
        z = dinv_ref[...] * (S_ref[0] + S_ref[1] + hp_ref[...]) + b_ref[...]
        z = jnp.maximum(z, 0.0)
        o_ref[...] = dinv_ref[...] * jnp.dot(
            z, w_ref[...], preferred_element_type=jnp.float32
        )

    return pl.pallas_call(
        body,
        grid=(N // R,),
        in_specs=[
            pl.BlockSpec((NC, R, H), lambda i: (0, i, 0)),
            pl.BlockSpec((R, H), lambda i: (i, 0)),
            pl.BlockSpec((R, H), lambda i: (i, 0)),
            pl.BlockSpec((1, H), lambda i: (0, 0)),
            pl.BlockSpec((H, H), lambda i: (0, 0)),
        ],
        out_specs=pl.BlockSpec((R, H), lambda i: (i, 0)),
        out_shape=jax.ShapeDtypeStruct((N, H), jnp.float32),
    )(S1, hp1, dinv, b1, W2)


def _tc_final(S2, hp2, dinv, b2, batch2, W3, b3, W4, b4):
    def body(
        S_ref, hp_ref, dinv_ref, b2_ref, bat_ref, w3_ref, b3_ref, w4_ref,
        b4_ref, o_ref, psum, cnt,
    ):
        i = pl.program_id(0)

        @pl.when(i == 0)
        def _():
            psum[...] = jnp.zeros((G, H), jnp.float32)
            cnt[...] = jnp.zeros((G, 1), jnp.float32)

        z = dinv_ref[...] * (S_ref[0] + S_ref[1] + hp_ref[...]) + b2_ref[...]
        z = jnp.maximum(z, 0.0)
        gid = lax.broadcasted_iota(jnp.int32, (R, G), 1)
        onehot = (bat_ref[...] == gid).astype(jnp.float32)
        psum[...] += lax.dot_general(
            onehot, z, (((0,), (0,)), ((), ())),
            preferred_element_type=jnp.float32,
            precision=lax.Precision.HIGHEST,
        )
        cnt[...] += lax.dot_general(
            onehot, jnp.ones((R, 1), jnp.float32), (((0,), (0,)), ((), ())),
            preferred_element_type=jnp.float32,
        )

        @pl.when(i == pl.num_programs(0) - 1)
        def _():
            p = psum[...] / jnp.maximum(cnt[...], 1.0)
            r = jnp.maximum(
                jnp.dot(p, w3_ref[...], preferred_element_type=jnp.float32)
                + b3_ref[...],
                0.0,
            )
            o_ref[...] = (
                jnp.dot(r, w4_ref[...], preferred_element_type=jnp.float32)
                + b4_ref[...]
            )

    return pl.pallas_call(
        body,
        grid=(N // R,),
        in_specs=[
            pl.BlockSpec((NC, R, H), lambda i: (0, i, 0)),
            pl.BlockSpec((R, H), lambda i: (i, 0)),
            pl.BlockSpec((R, H), lambda i: (i, 0)),
            pl.BlockSpec((1, H), lambda i: (0, 0)),
            pl.BlockSpec((R, 1), lambda i: (i, 0)),
            pl.BlockSpec((H, 128), lambda i: (0, 0)),
            pl.BlockSpec((1, 128), lambda i: (0, 0)),
            pl.BlockSpec((128, 1), lambda i: (0, 0)),
            pl.BlockSpec((1, 1), lambda i: (0, 0)),
        ],
        out_specs=pl.BlockSpec((G, 1), lambda i: (0, 0)),
        out_shape=jax.ShapeDtypeStruct((G, 1), jnp.float32),
        scratch_shapes=[
            pltpu.VMEM((G, H), jnp.float32),
            pltpu.VMEM((G, 1), jnp.float32),
        ],
    )(S2, hp2, dinv, b2, batch2, W3, b3, W4, b4)


def kernel(x, edge_index, batch, W1, b1, W2, b2, W3, b3, W4, b4):
    src3 = edge_index[0].reshape(NW, NCHUNK, CHUNK)
    dst3 = edge_index[1].reshape(NW, NCHUNK, CHUNK)
    zeros16 = jnp.zeros((NA, 16), jnp.float32)
    zeros64 = jnp.zeros((NA, H), jnp.float32)
    batch2 = batch.reshape(N, 1)
    b1r = b1.reshape(1, H)
    b2r = b2.reshape(1, H)
    b3r = b3.reshape(1, 128)
    b4r = b4.reshape(1, 1)

    degp = _sc_degree(dst3, zeros16)
    dinv, hp1 = _tc_prep(degp, x, W1)
    S1 = _sc_aggregate(hp1, src3, dst3, zeros64)
    hp2 = _tc_mid(S1, hp1, dinv, b1r, W2)
    S2 = _sc_aggregate(hp2, src3, dst3, zeros64)
    out = _tc_final(S2, hp2, dinv, b2r, batch2, W3, b3r, W4, b4r)
    return jnp.squeeze(out)
